# baseline (speedup 1.0000x reference)
import functools

import jax
import jax.numpy as jnp
from jax import lax
from jax.experimental import pallas as pl
from jax.experimental.pallas import tpu as pltpu
from jax.experimental.pallas import tpu_sc as plsc

N = 10000
E = 320000
D = 128
S = 4
ORDER = 4

NC = 2
NS = 16
NW = NC * NS
C = 128
KCH = 81
EPT = KCH * C
E_PAD = NW * EPT
PAD = E_PAD - E
N_ACC = 10112
RPT = N_ACC // NS
N_DUMMY = N_ACC - N

_mesh = plsc.VectorSubcoreMesh(core_axis_name="c", subcore_axis_name="s",
                               num_cores=NC, num_subcores=NS)


LANE_IN = 64


@functools.partial(
    pl.kernel,
    out_type=jax.ShapeDtypeStruct((NC, N_ACC, D), jnp.float32),
    mesh=_mesh,
    scratch_types=[
        pltpu.VMEM((KCH, C), jnp.int32),
        pltpu.VMEM((KCH, C), jnp.int32),
        pltpu.VMEM((C, D), jnp.float32),
        pltpu.VMEM_SHARED((N_ACC, D), jnp.float32),
        pltpu.SemaphoreType.DMA,
    ],
)
def _sc_degrees(src_hbm, dst_hbm, onesa_hbm, onesb_hbm, ztile_hbm,
                deg_hbm, sidx, didx, buf, acc, dsem):
    c = lax.axis_index("c")
    s = lax.axis_index("s")
    w = c * NS + s
    for z in range(RPT // C):
        pltpu.sync_copy(ztile_hbm, acc.at[pl.ds(s * RPT + z * C, C)])
    if RPT % C:
        pltpu.sync_copy(ztile_hbm.at[pl.ds(0, RPT % C)],
                        acc.at[pl.ds(s * RPT + (RPT // C) * C, RPT % C)])
    pltpu.sync_copy(src_hbm.at[w], sidx)
    pltpu.sync_copy(dst_hbm.at[w], didx)
    pltpu.sync_copy(onesa_hbm, buf)
    plsc.subcore_barrier()

    @pl.loop(0, KCH)
    def _(j):
        pltpu.async_copy(buf, acc.at[sidx.at[j]], dsem, add=True)

    @pl.loop(0, KCH)
    def _(j):
        pltpu.make_async_copy(onesa_hbm, buf, dsem).wait()

    pltpu.sync_copy(onesb_hbm, buf)

    @pl.loop(0, KCH)
    def _(j):
        pltpu.async_copy(buf, acc.at[didx.at[j]], dsem, add=True)

    @pl.loop(0, KCH)
    def _(j):
        pltpu.make_async_copy(onesa_hbm, buf, dsem).wait()

    plsc.subcore_barrier()
    pltpu.sync_copy(acc.at[pl.ds(s * RPT, RPT)],
                    deg_hbm.at[c, pl.ds(s * RPT, RPT)])


@functools.partial(
    pl.kernel,
    out_type=jax.ShapeDtypeStruct((NC, N_ACC, D), jnp.float32),
    mesh=_mesh,
    scratch_types=[
        pltpu.VMEM((C,), jnp.int32),
        pltpu.VMEM((C,), jnp.int32),
        pltpu.VMEM((C,), jnp.int32),
        pltpu.VMEM((C,), jnp.int32),
        pltpu.VMEM((C,), jnp.int32),
        pltpu.VMEM((C,), jnp.int32),
        pltpu.VMEM((C, D), jnp.float32),
        pltpu.VMEM((C, D), jnp.float32),
        pltpu.VMEM((C, D), jnp.float32),
        pltpu.VMEM_SHARED((N_ACC, D), jnp.float32),
        pltpu.SemaphoreType.DMA,
        pltpu.SemaphoreType.DMA,
        pltpu.SemaphoreType.DMA,
        pltpu.SemaphoreType.DMA,
        pltpu.SemaphoreType.DMA,
        pltpu.SemaphoreType.DMA,
        pltpu.SemaphoreType.DMA,
        pltpu.SemaphoreType.DMA,
        pltpu.SemaphoreType.DMA,
        pltpu.SemaphoreType.DMA,
        pltpu.SemaphoreType.DMA,
        pltpu.SemaphoreType.DMA,
    ],
)
def _sc_hop(u_hbm, src_hbm, dst_hbm, ztile_hbm, part_hbm,
            sib0, sib1, sib2, dib0, dib1, dib2, gb0, gb1, gb2, acc,
            is0, is1, is2, id0, id1, id2, gs0, gs1, gs2, ss0, ss1, ss2):
    c = lax.axis_index("c")
    s = lax.axis_index("s")
    w = c * NS + s
    sib = (sib0, sib1, sib2)
    dib = (dib0, dib1, dib2)
    gb = (gb0, gb1, gb2)
    isem = (is0, is1, is2)
    idsem = (id0, id1, id2)
    gsem = (gs0, gs1, gs2)
    ssem = (ss0, ss1, ss2)
    for z in range(RPT // C):
        pltpu.sync_copy(ztile_hbm, acc.at[pl.ds(s * RPT + z * C, C)])
    if RPT % C:
        pltpu.sync_copy(ztile_hbm.at[pl.ds(0, RPT % C)],
                        acc.at[pl.ds(s * RPT + (RPT // C) * C, RPT % C)])
    plsc.subcore_barrier()

    def fire_sidx(j, b):
        pltpu.async_copy(src_hbm.at[w, j], sib[b], isem[b])

    def fire_didx(j, b):
        pltpu.async_copy(dst_hbm.at[w, j], dib[b], idsem[b])

    def fire_gather(b):
        pltpu.async_copy(u_hbm.at[sib[b]], gb[b], gsem[b])

    def fire_scatter(b):
        pltpu.async_copy(gb[b], acc.at[dib[b]], ssem[b], add=True)

    def wait_idx(sem, ref):
        pltpu.make_async_copy(src_hbm.at[w, 0], ref, sem).wait()

    def wait_buf(sem, b):
        pltpu.make_async_copy(ztile_hbm, gb[b], sem).wait()

    fire_sidx(0, 0)
    fire_sidx(1, 1)
    fire_sidx(2, 2)
    fire_didx(0, 0)
    fire_didx(1, 1)
    wait_idx(isem[0], sib[0])
    fire_gather(0)
    wait_idx(isem[1], sib[1])
    fire_gather(1)
    fire_didx(2, 2)
    wait_idx(isem[2], sib[2])
    fire_gather(2)
    wait_buf(gsem[0], 0)
    wait_idx(idsem[0], dib[0])
    fire_scatter(0)
    fire_sidx(3, 0)

    @pl.loop(1, KCH // 3)
    def _(t):
        j0 = 3 * t
        for b in range(3):
            j = j0 + b
            c2 = (b + 1) % 3
            pltpu.make_async_copy(ztile_hbm, gb[c2], gsem[c2]).wait()
            pltpu.make_async_copy(src_hbm.at[w, 0], dib[c2],
                                  idsem[c2]).wait()
            pltpu.async_copy(gb[c2], acc.at[dib[c2]], ssem[c2], add=True)
            jn = jnp.where(j + 1 < KCH, j + 1, 0)
            pltpu.async_copy(src_hbm.at[w, jn], sib[c2], isem[c2])
            pltpu.make_async_copy(ztile_hbm, gb[b], ssem[b]).wait()
            pltpu.async_copy(dst_hbm.at[w, j], dib[b], idsem[b])
            pltpu.make_async_copy(src_hbm.at[w, 0], sib[b], isem[b]).wait()
            pltpu.async_copy(u_hbm.at[sib[b]], gb[b], gsem[b])

    bL2 = (KCH - 2) % 3
    bL1 = (KCH - 1) % 3
    wait_buf(gsem[bL2], bL2)
    wait_idx(idsem[bL2], dib[bL2])
    fire_scatter(bL2)
    wait_buf(gsem[bL1], bL1)
    wait_idx(idsem[bL1], dib[bL1])
    fire_scatter(bL1)
    for b in range(3):
        pltpu.make_async_copy(ztile_hbm, gb[b], ssem[b]).wait()
    wait_idx(isem[(KCH) % 3], sib[(KCH) % 3])
    plsc.subcore_barrier()
    pltpu.sync_copy(acc.at[pl.ds(s * RPT, RPT)],
                    part_hbm.at[c, pl.ds(s * RPT, RPT)])


_R2 = 1264
_RF = 2000


def _prep_body(deg_ref, x_ref, u0_ref, p_ref, di_ref):
    do = deg_ref[0, :, 0:1] + deg_ref[1, :, 0:1]
    dI = deg_ref[0, :, LANE_IN:LANE_IN + 1] + deg_ref[1, :, LANE_IN:LANE_IN + 1]
    rdo = lax.rsqrt(jnp.maximum(do, 1.0))
    rdi = lax.rsqrt(jnp.maximum(dI, 1.0))
    u0_ref[...] = x_ref[...] * rdo
    p_ref[...] = rdo * rdi
    di_ref[...] = rdi


_tc_prep = pl.pallas_call(
    _prep_body,
    grid=(N_ACC // _R2,),
    in_specs=[
        pl.BlockSpec((NC, _R2, D), lambda i: (0, i, 0)),
        pl.BlockSpec((_R2, D), lambda i: (i, 0)),
    ],
    out_specs=[
        pl.BlockSpec((_R2, D), lambda i: (i, 0)),
        pl.BlockSpec((_R2, 1), lambda i: (i, 0)),
        pl.BlockSpec((_R2, 1), lambda i: (i, 0)),
    ],
    out_shape=[
        jax.ShapeDtypeStruct((N_ACC, D), jnp.float32),
        jax.ShapeDtypeStruct((N_ACC, 1), jnp.float32),
        jax.ShapeDtypeStruct((N_ACC, 1), jnp.float32),
    ],
)


def _comb_body(part_ref, p_ref, v_ref, u_ref):
    v = part_ref[0] + part_ref[1]
    v_ref[...] = v
    u_ref[...] = v * p_ref[...]


_tc_combine = pl.pallas_call(
    _comb_body,
    grid=(N_ACC // _R2,),
    in_specs=[
        pl.BlockSpec((NC, _R2, D), lambda i: (0, i, 0)),
        pl.BlockSpec((_R2, 1), lambda i: (i, 0)),
    ],
    out_specs=[
        pl.BlockSpec((_R2, D), lambda i: (i, 0)),
        pl.BlockSpec((_R2, D), lambda i: (i, 0)),
    ],
    out_shape=[
        jax.ShapeDtypeStruct((N_ACC, D), jnp.float32),
        jax.ShapeDtypeStruct((N_ACC, D), jnp.float32),
    ],
)


def _final_body(co_ref, x_ref, v1_ref, v2_ref, v3_ref, p4_ref, di_ref,
                out_ref):
    di_col = di_ref[...]
    vs = (v1_ref[...], v2_ref[...], v3_ref[...], p4_ref[0] + p4_ref[1])
    for si in range(S):
        acc = co_ref[si, 1] * vs[0]
        for k in range(1, ORDER):
            acc = acc + co_ref[si, k + 1] * vs[k]
        out_ref[si] = co_ref[si, 0] * x_ref[...] + di_col * acc


_tc_final = pl.pallas_call(
    _final_body,
    grid=(N // _RF,),
    in_specs=[
        pl.BlockSpec(memory_space=pltpu.SMEM),
        pl.BlockSpec((_RF, D), lambda i: (i, 0)),
        pl.BlockSpec((_RF, D), lambda i: (i, 0)),
        pl.BlockSpec((_RF, D), lambda i: (i, 0)),
        pl.BlockSpec((_RF, D), lambda i: (i, 0)),
        pl.BlockSpec((NC, _RF, D), lambda i: (0, i, 0)),
        pl.BlockSpec((_RF, 1), lambda i: (i, 0)),
    ],
    out_specs=pl.BlockSpec((S, _RF, D), lambda i: (0, i, 0)),
    out_shape=jax.ShapeDtypeStruct((S, N, D), jnp.float32),
)


def kernel(x, edge_index, coeffs):
    src = edge_index[0].astype(jnp.int32)
    dst = edge_index[1].astype(jnp.int32)

    pad_iota = jnp.arange(PAD, dtype=jnp.int32)
    dummy = N + pad_iota % N_DUMMY
    src_g = jnp.concatenate([src, pad_iota % N]).reshape(NW, KCH, C)
    dst_p = jnp.concatenate([dst, dummy]).reshape(NW, KCH, C)
    src_d = jnp.concatenate([src, dummy]).reshape(NW, KCH, C)

    x_p = jnp.pad(x, ((0, N_ACC - N), (0, 0)))
    lane = lax.broadcasted_iota(jnp.int32, (C, D), 1)
    onesa = jnp.where(lane == 0, 1.0, 0.0).astype(jnp.float32)
    onesb = jnp.where(lane == LANE_IN, 1.0, 0.0).astype(jnp.float32)
    ztile = jnp.zeros((C, D), jnp.float32)

    deg_p = _sc_degrees(src_d, dst_p, onesa, onesb, ztile)
    u, p, di = _tc_prep(deg_p, x_p)

    vs = []
    for k in range(ORDER):
        part = _sc_hop(u, src_g, dst_p, ztile)
        if k < ORDER - 1:
            v, u = _tc_combine(part, p)
            vs.append(v)

    return _tc_final(coeffs, x_p, vs[0], vs[1], vs[2], part, di)


# speedup vs baseline: 1.0526x; 1.0526x over previous
import functools

import jax
import jax.numpy as jnp
from jax import lax
from jax.experimental import pallas as pl
from jax.experimental.pallas import tpu as pltpu
from jax.experimental.pallas import tpu_sc as plsc

N = 10000
E = 320000
D = 128
S = 4
ORDER = 4

NC = 2
NS = 16
NW = NC * NS
C = 128
KCH = 81
EPT = KCH * C
E_PAD = NW * EPT
PAD = E_PAD - E
N_ACC = 10112
RPT = N_ACC // NS
N_DUMMY = N_ACC - N

_mesh = plsc.VectorSubcoreMesh(core_axis_name="c", subcore_axis_name="s",
                               num_cores=NC, num_subcores=NS)


LANE_IN = 64


@functools.partial(
    pl.kernel,
    out_type=jax.ShapeDtypeStruct((NC, N_ACC, D), jnp.float32),
    mesh=_mesh,
    scratch_types=[
        pltpu.VMEM((KCH, C), jnp.int32),
        pltpu.VMEM((KCH, C), jnp.int32),
        pltpu.VMEM((C, D), jnp.float32),
        pltpu.VMEM_SHARED((N_ACC, D), jnp.float32),
        pltpu.SemaphoreType.DMA,
    ],
)
def _sc_degrees(src_hbm, dst_hbm, onesa_hbm, onesb_hbm, ztile_hbm,
                deg_hbm, sidx, didx, buf, acc, dsem):
    c = lax.axis_index("c")
    s = lax.axis_index("s")
    w = c * NS + s
    for z in range(RPT // C):
        pltpu.sync_copy(ztile_hbm, acc.at[pl.ds(s * RPT + z * C, C)])
    if RPT % C:
        pltpu.sync_copy(ztile_hbm.at[pl.ds(0, RPT % C)],
                        acc.at[pl.ds(s * RPT + (RPT // C) * C, RPT % C)])
    pltpu.sync_copy(src_hbm.at[w], sidx)
    pltpu.sync_copy(dst_hbm.at[w], didx)
    pltpu.sync_copy(onesa_hbm, buf)
    plsc.subcore_barrier()

    @pl.loop(0, KCH)
    def _(j):
        pltpu.async_copy(buf, acc.at[sidx.at[j]], dsem, add=True)

    @pl.loop(0, KCH)
    def _(j):
        pltpu.make_async_copy(onesa_hbm, buf, dsem).wait()

    pltpu.sync_copy(onesb_hbm, buf)

    @pl.loop(0, KCH)
    def _(j):
        pltpu.async_copy(buf, acc.at[didx.at[j]], dsem, add=True)

    @pl.loop(0, KCH)
    def _(j):
        pltpu.make_async_copy(onesa_hbm, buf, dsem).wait()

    plsc.subcore_barrier()
    pltpu.sync_copy(acc.at[pl.ds(s * RPT, RPT)],
                    deg_hbm.at[c, pl.ds(s * RPT, RPT)])


@functools.partial(
    pl.kernel,
    out_type=jax.ShapeDtypeStruct((NC, N_ACC, D), jnp.float32),
    mesh=_mesh,
    scratch_types=[
        pltpu.VMEM((C,), jnp.int32),
        pltpu.VMEM((C,), jnp.int32),
        pltpu.VMEM((C,), jnp.int32),
        pltpu.VMEM((C,), jnp.int32),
        pltpu.VMEM((C,), jnp.int32),
        pltpu.VMEM((C,), jnp.int32),
        pltpu.VMEM((C, D), jnp.float32),
        pltpu.VMEM((C, D), jnp.float32),
        pltpu.VMEM((C, D), jnp.float32),
        pltpu.VMEM_SHARED((N_ACC, D), jnp.float32),
        pltpu.SemaphoreType.DMA,
        pltpu.SemaphoreType.DMA,
        pltpu.SemaphoreType.DMA,
        pltpu.SemaphoreType.DMA,
        pltpu.SemaphoreType.DMA,
        pltpu.SemaphoreType.DMA,
        pltpu.SemaphoreType.DMA,
        pltpu.SemaphoreType.DMA,
        pltpu.SemaphoreType.DMA,
        pltpu.SemaphoreType.DMA,
        pltpu.SemaphoreType.DMA,
        pltpu.SemaphoreType.DMA,
    ],
)
def _sc_hop(u_hbm, src_hbm, dst_hbm, ztile_hbm, part_hbm,
            sib0, sib1, sib2, dib0, dib1, dib2, gb0, gb1, gb2, acc,
            is0, is1, is2, id0, id1, id2, gs0, gs1, gs2, ss0, ss1, ss2):
    c = lax.axis_index("c")
    s = lax.axis_index("s")
    w = c * NS + s
    sib = (sib0, sib1, sib2)
    dib = (dib0, dib1, dib2)
    gb = (gb0, gb1, gb2)
    isem = (is0, is1, is2)
    idsem = (id0, id1, id2)
    gsem = (gs0, gs1, gs2)
    ssem = (ss0, ss1, ss2)
    for z in range(RPT // C):
        pltpu.sync_copy(ztile_hbm, acc.at[pl.ds(s * RPT + z * C, C)])
    if RPT % C:
        pltpu.sync_copy(ztile_hbm.at[pl.ds(0, RPT % C)],
                        acc.at[pl.ds(s * RPT + (RPT // C) * C, RPT % C)])
    plsc.subcore_barrier()

    def fire_sidx(j, b):
        pltpu.async_copy(src_hbm.at[w, j], sib[b], isem[b])

    def fire_didx(j, b):
        pltpu.async_copy(dst_hbm.at[w, j], dib[b], idsem[b])

    def fire_gather(b):
        pltpu.async_copy(u_hbm.at[sib[b]], gb[b], gsem[b])

    def fire_scatter(b):
        pltpu.async_copy(gb[b], acc.at[dib[b]], ssem[b], add=True)

    def wait_idx(sem, ref):
        pltpu.make_async_copy(src_hbm.at[w, 0], ref, sem).wait()

    def wait_buf(sem, b):
        pltpu.make_async_copy(ztile_hbm, gb[b], sem).wait()

    fire_sidx(0, 0)
    fire_sidx(1, 1)
    fire_sidx(2, 2)
    fire_didx(0, 0)
    fire_didx(1, 1)
    wait_idx(isem[0], sib[0])
    fire_gather(0)
    wait_idx(isem[1], sib[1])
    fire_gather(1)
    fire_didx(2, 2)
    wait_idx(isem[2], sib[2])
    fire_gather(2)
    wait_buf(gsem[0], 0)
    wait_idx(idsem[0], dib[0])
    fire_scatter(0)
    fire_sidx(3, 0)

    @pl.loop(1, KCH // 3)
    def _(t):
        j0 = 3 * t
        for b in range(3):
            j = j0 + b
            c2 = (b + 1) % 3
            pltpu.make_async_copy(ztile_hbm, gb[b], ssem[b]).wait()
            pltpu.async_copy(dst_hbm.at[w, j], dib[b], idsem[b])
            pltpu.make_async_copy(src_hbm.at[w, 0], sib[b], isem[b]).wait()
            pltpu.async_copy(u_hbm.at[sib[b]], gb[b], gsem[b])
            pltpu.make_async_copy(ztile_hbm, gb[c2], gsem[c2]).wait()
            pltpu.make_async_copy(src_hbm.at[w, 0], dib[c2],
                                  idsem[c2]).wait()
            pltpu.async_copy(gb[c2], acc.at[dib[c2]], ssem[c2], add=True)
            jn = jnp.where(j + 1 < KCH, j + 1, 0)
            pltpu.async_copy(src_hbm.at[w, jn], sib[c2], isem[c2])

    bL2 = (KCH - 2) % 3
    bL1 = (KCH - 1) % 3
    wait_buf(gsem[bL2], bL2)
    wait_idx(idsem[bL2], dib[bL2])
    fire_scatter(bL2)
    wait_buf(gsem[bL1], bL1)
    wait_idx(idsem[bL1], dib[bL1])
    fire_scatter(bL1)
    for b in range(3):
        pltpu.make_async_copy(ztile_hbm, gb[b], ssem[b]).wait()
    wait_idx(isem[(KCH) % 3], sib[(KCH) % 3])
    plsc.subcore_barrier()
    pltpu.sync_copy(acc.at[pl.ds(s * RPT, RPT)],
                    part_hbm.at[c, pl.ds(s * RPT, RPT)])


_R2 = 1264
_RF = 2000


def _prep_body(deg_ref, x_ref, u0_ref, p_ref, di_ref):
    do = deg_ref[0, :, 0:1] + deg_ref[1, :, 0:1]
    dI = deg_ref[0, :, LANE_IN:LANE_IN + 1] + deg_ref[1, :, LANE_IN:LANE_IN + 1]
    rdo = lax.rsqrt(jnp.maximum(do, 1.0))
    rdi = lax.rsqrt(jnp.maximum(dI, 1.0))
    u0_ref[...] = x_ref[...] * rdo
    p_ref[...] = rdo * rdi
    di_ref[...] = rdi


_tc_prep = pl.pallas_call(
    _prep_body,
    grid=(N_ACC // _R2,),
    in_specs=[
        pl.BlockSpec((NC, _R2, D), lambda i: (0, i, 0)),
        pl.BlockSpec((_R2, D), lambda i: (i, 0)),
    ],
    out_specs=[
        pl.BlockSpec((_R2, D), lambda i: (i, 0)),
        pl.BlockSpec((_R2, 1), lambda i: (i, 0)),
        pl.BlockSpec((_R2, 1), lambda i: (i, 0)),
    ],
    out_shape=[
        jax.ShapeDtypeStruct((N_ACC, D), jnp.float32),
        jax.ShapeDtypeStruct((N_ACC, 1), jnp.float32),
        jax.ShapeDtypeStruct((N_ACC, 1), jnp.float32),
    ],
)


def _comb_body(part_ref, p_ref, v_ref, u_ref):
    v = part_ref[0] + part_ref[1]
    v_ref[...] = v
    u_ref[...] = v * p_ref[...]


_tc_combine = pl.pallas_call(
    _comb_body,
    grid=(N_ACC // _R2,),
    in_specs=[
        pl.BlockSpec((NC, _R2, D), lambda i: (0, i, 0)),
        pl.BlockSpec((_R2, 1), lambda i: (i, 0)),
    ],
    out_specs=[
        pl.BlockSpec((_R2, D), lambda i: (i, 0)),
        pl.BlockSpec((_R2, D), lambda i: (i, 0)),
    ],
    out_shape=[
        jax.ShapeDtypeStruct((N_ACC, D), jnp.float32),
        jax.ShapeDtypeStruct((N_ACC, D), jnp.float32),
    ],
)


def _final_body(co_ref, x_ref, v1_ref, v2_ref, v3_ref, p4_ref, di_ref,
                out_ref):
    di_col = di_ref[...]
    vs = (v1_ref[...], v2_ref[...], v3_ref[...], p4_ref[0] + p4_ref[1])
    for si in range(S):
        acc = co_ref[si, 1] * vs[0]
        for k in range(1, ORDER):
            acc = acc + co_ref[si, k + 1] * vs[k]
        out_ref[si] = co_ref[si, 0] * x_ref[...] + di_col * acc


_tc_final = pl.pallas_call(
    _final_body,
    grid=(N // _RF,),
    in_specs=[
        pl.BlockSpec(memory_space=pltpu.SMEM),
        pl.BlockSpec((_RF, D), lambda i: (i, 0)),
        pl.BlockSpec((_RF, D), lambda i: (i, 0)),
        pl.BlockSpec((_RF, D), lambda i: (i, 0)),
        pl.BlockSpec((_RF, D), lambda i: (i, 0)),
        pl.BlockSpec((NC, _RF, D), lambda i: (0, i, 0)),
        pl.BlockSpec((_RF, 1), lambda i: (i, 0)),
    ],
    out_specs=pl.BlockSpec((S, _RF, D), lambda i: (0, i, 0)),
    out_shape=jax.ShapeDtypeStruct((S, N, D), jnp.float32),
)


def kernel(x, edge_index, coeffs):
    src = edge_index[0].astype(jnp.int32)
    dst = edge_index[1].astype(jnp.int32)

    pad_iota = jnp.arange(PAD, dtype=jnp.int32)
    dummy = N + pad_iota % N_DUMMY
    src_g = jnp.concatenate([src, pad_iota % N]).reshape(NW, KCH, C)
    dst_p = jnp.concatenate([dst, dummy]).reshape(NW, KCH, C)
    src_d = jnp.concatenate([src, dummy]).reshape(NW, KCH, C)

    x_p = jnp.pad(x, ((0, N_ACC - N), (0, 0)))
    lane = lax.broadcasted_iota(jnp.int32, (C, D), 1)
    onesa = jnp.where(lane == 0, 1.0, 0.0).astype(jnp.float32)
    onesb = jnp.where(lane == LANE_IN, 1.0, 0.0).astype(jnp.float32)
    ztile = jnp.zeros((C, D), jnp.float32)

    deg_p = _sc_degrees(src_d, dst_p, onesa, onesb, ztile)
    u, p, di = _tc_prep(deg_p, x_p)

    vs = []
    for k in range(ORDER):
        part = _sc_hop(u, src_g, dst_p, ztile)
        if k < ORDER - 1:
            v, u = _tc_combine(part, p)
            vs.append(v)

    return _tc_final(coeffs, x_p, vs[0], vs[1], vs[2], part, di)
